# Initial kernel scaffold; baseline (speedup 1.0000x reference)
import jax
import jax.numpy as jnp
from jax.experimental import pallas as pl


def kernel(x, edge_index, W1, b1, W2, b2, W3, b3, Wl, bl):
    raise NotImplementedError("write your pallas kernel here")


# speedup vs baseline: 2.8989x; 2.8989x over previous
import jax
import jax.numpy as jnp
from jax import lax
from jax.experimental import pallas as pl

N = 10000
E = 320000
F_IN = 128
HID = 64
N_CLS = 64


def _tc1_body(x_ref, w_ref, deg_ref, g_ref, dinv_ref):
    dinv = lax.rsqrt(deg_ref[...] + 1.0)
    h = jnp.dot(x_ref[...], w_ref[...], preferred_element_type=jnp.float32)
    g_ref[...] = h * dinv
    dinv_ref[...] = dinv


_tc1 = pl.pallas_call(
    _tc1_body,
    out_shape=[
        jax.ShapeDtypeStruct((N, HID), jnp.float32),
        jax.ShapeDtypeStruct((N, 1), jnp.float32),
    ],
)


def _tcmid_body(p_ref, g_ref, dinv_ref, b_ref, w_ref, o_ref):
    dinv = dinv_ref[...]
    u = (p_ref[...] + g_ref[...]) * dinv + b_ref[...][None, :]
    a = jnp.where(u > 0, u, jnp.exp(jnp.minimum(u, 0.0)) - 1.0)
    h = jnp.dot(a, w_ref[...], preferred_element_type=jnp.float32)
    o_ref[...] = h * dinv


_tcmid = pl.pallas_call(
    _tcmid_body,
    out_shape=jax.ShapeDtypeStruct((N, HID), jnp.float32),
)


def _tcfin_body(p_ref, g_ref, dinv_ref, b_ref, wl_ref, bl_ref, o_ref):
    u = (p_ref[...] + g_ref[...]) * dinv_ref[...] + b_ref[...][None, :]
    o_ref[...] = (jnp.dot(u, wl_ref[...], preferred_element_type=jnp.float32)
                  + bl_ref[...][None, :])


_tcfin = pl.pallas_call(
    _tcfin_body,
    out_shape=jax.ShapeDtypeStruct((N, N_CLS), jnp.float32),
)


def kernel(x, edge_index, W1, b1, W2, b2, W3, b3, Wl, bl):
    e32 = edge_index.astype(jnp.int32)
    src, dst = e32[0], e32[1]

    deg = jnp.zeros((N,), jnp.float32).at[dst].add(1.0)[:, None]

    def agg(g):
        return jnp.zeros((N, HID), jnp.float32).at[dst].add(g[src])

    g1, dinv = _tc1(x, W1, deg)
    g2 = _tcmid(agg(g1), g1, dinv, b1, W2)
    g3 = _tcmid(agg(g2), g2, dinv, b2, W3)
    return _tcfin(agg(g3), g3, dinv, b3, Wl, bl)
